# baseline (speedup 1.0000x reference)
import functools

import jax
import jax.numpy as jnp
from jax import lax
from jax.experimental import pallas as pl
from jax.experimental.pallas import tpu as pltpu
from jax.experimental.pallas import tpu_sc as plsc

_L = 16


def _make_sc_embed(B, L, V, D):
    NW = 32
    assert B % NW == 0
    bpw = B // NW
    nd = D // _L
    cA, cB = 104, L - 104
    RA = 112
    RT = RA + cB
    assert cA % 8 == 0 and cA <= 128 and 0 < cB <= 128 and cB % _L == 0

    mesh = plsc.VectorSubcoreMesh(core_axis_name="c", subcore_axis_name="s")

    @functools.partial(
        pl.kernel,
        mesh=mesh,
        compiler_params=pltpu.CompilerParams(use_tc_tiling_on_sc=True),
        out_type=jax.ShapeDtypeStruct((B, L, D), jnp.float32),
        scratch_types=[
            pltpu.VMEM((2, RA), jnp.int32),
            pltpu.VMEM((2, RA), jnp.float32),
            pltpu.VMEM((2, RA), jnp.int32),
            pltpu.VMEM((RT * D,), jnp.float32),
            pltpu.VMEM((2 * D,), jnp.float32),
            pltpu.VMEM((RT, 2 * D), jnp.float32),
            pltpu.VMEM((RT, D), jnp.float32),
            pltpu.SemaphoreType.DMA,
        ],
    )
    def sc_embed(idx_hbm, mask_hbm, table_hbm, q_hbm, me_hbm, out_hbm,
                 idx_v, mask_v, pidx_v, q_v, me_v, buf, obuf, sem):
        wid = lax.axis_index("s") * 2 + lax.axis_index("c")
        pltpu.sync_copy(q_hbm, q_v)
        pltpu.sync_copy(me_hbm, me_v)
        emb = [me_v[pl.ds(_L * j, _L)] for j in range(nd)]

        def chunk_body(g, _):
            bb = wid * bpw + g
            pltpu.sync_copy(idx_hbm.at[bb], idx_v)
            pltpu.sync_copy(mask_hbm.at[bb], mask_v)
            for h in range(2):
                for t in range(RA // _L):
                    sl = pl.ds(t * _L, _L)
                    pidx_v[h, sl] = lax.shift_right_logical(idx_v[h, sl], 1)
            cp0 = pltpu.async_copy(
                table_hbm.at[pidx_v.at[0, pl.ds(0, cA)]],
                buf.at[pl.ds(0, cA)], sem)
            cp1 = pltpu.async_copy(
                table_hbm.at[pidx_v.at[1, pl.ds(0, cB)]],
                buf.at[pl.ds(RA, cB)], sem)
            cp0.wait()
            cp1.wait()

            for h, ngrp in ((0, RA // _L), (1, cB // _L)):
                def grp_body(t, _, h=h):
                    base = t * _L
                    m16 = mask_v[h, pl.ds(base, _L)]
                    i16 = idx_v[h, pl.ds(base, _L)]
                    for r in range(_L):
                        mr = jnp.full((_L,), m16[r], jnp.float32)
                        off = (i16[r] & 1) * D
                        l = h * RA + base + r
                        me = [mr * emb[j] for j in range(nd)]
                        for j in range(nd):
                            g16 = buf[l, pl.ds(off + _L * j, _L)]
                            q16 = q_v[pl.ds(l * D + _L * j, _L)]
                            obuf[l, pl.ds(_L * j, _L)] = mr * g16 + q16 - me[j]
                    return 0

                lax.fori_loop(0, ngrp, grp_body, 0)

            pltpu.sync_copy(obuf.at[pl.ds(0, cA)],
                            out_hbm.at[bb, pl.ds(0, cA)])
            pltpu.sync_copy(obuf.at[pl.ds(RA, cB)],
                            out_hbm.at[bb, pl.ds(cA, cB)])
            return 0

        lax.fori_loop(0, bpw, chunk_body, 0)

    return sc_embed


def _pack_regions(x, cA, RA, dtype):
    B, L = x.shape
    cB = L - cA
    a = jnp.pad(x[:, :cA].astype(dtype), ((0, 0), (0, RA - cA)))
    b = jnp.pad(x[:, cA:].astype(dtype), ((0, 0), (0, RA - cB)))
    return jnp.stack([a, b], axis=1)


def kernel(item_id, token_mask, item_table, mask_emb, pos_emb):
    B, L = item_id.shape
    V, D = item_table.shape
    cA, RA = 104, 112
    RT = RA + (L - cA)
    idxp = _pack_regions(item_id, cA, RA, jnp.int32)
    maskp = _pack_regions(token_mask, cA, RA, jnp.float32)
    table2 = item_table.reshape(V // 2, 2 * D)
    q = pos_emb + mask_emb
    qp = jnp.concatenate(
        [q[:cA], jnp.zeros((RA - cA, D), jnp.float32), q[cA:],
         jnp.zeros((RT - RA - (L - cA), D), jnp.float32)]).reshape(-1)
    mep = jnp.pad(mask_emb.reshape(-1), (0, D))
    sc_embed = _make_sc_embed(B, L, V, D)
    return sc_embed(idxp, maskp, table2, qp, mep)


# speedup vs baseline: 1.3528x; 1.3528x over previous
import functools

import jax
import jax.numpy as jnp
from jax import lax
from jax.experimental import pallas as pl
from jax.experimental.pallas import tpu as pltpu
from jax.experimental.pallas import tpu_sc as plsc

_L = 16


def _make_sc_embed(B, L, V, D):
    NW = 32
    assert B % NW == 0
    bpw = B // NW
    nd = D // _L
    cA, cB = 128, L - 128
    RT = cA + ((cB + _L - 1) // _L) * _L
    assert 0 < cB <= 128 and cA % 8 == 0

    mesh = plsc.VectorSubcoreMesh(core_axis_name="c", subcore_axis_name="s")

    @functools.partial(
        pl.kernel,
        mesh=mesh,
        compiler_params=pltpu.CompilerParams(use_tc_tiling_on_sc=False),
        out_type=jax.ShapeDtypeStruct((B, L, D), jnp.float32),
        scratch_types=[
            pltpu.VMEM((2, cA), jnp.int32),
            pltpu.VMEM((2, cA), jnp.float32),
            pltpu.VMEM((RT, D), jnp.float32),
            pltpu.VMEM((1, D), jnp.float32),
            pltpu.VMEM((RT, D), jnp.float32),
            pltpu.SemaphoreType.DMA,
        ],
    )
    def sc_embed(idx_hbm, mask_hbm, table_hbm, me_hbm, pos_hbm, out_hbm,
                 idx_v, mask_v, pos_v, me_v, buf, sem):
        wid = lax.axis_index("s") * 2 + lax.axis_index("c")
        pltpu.sync_copy(pos_hbm.at[pl.ds(0, cA)], pos_v.at[pl.ds(0, cA)])
        pltpu.sync_copy(pos_hbm.at[pl.ds(cA, cB)], pos_v.at[pl.ds(cA, cB)])
        pltpu.sync_copy(me_hbm, me_v)
        emb = [me_v[0, pl.ds(_L * j, _L)] for j in range(nd)]

        def chunk_body(g, _):
            bb = wid * bpw + g
            pltpu.sync_copy(idx_hbm.at[pl.ds(2 * bb, 2)], idx_v)
            pltpu.sync_copy(mask_hbm.at[pl.ds(2 * bb, 2)], mask_v)
            cp0 = pltpu.async_copy(
                table_hbm.at[idx_v.at[0]], buf.at[pl.ds(0, cA)], sem)
            cp1 = pltpu.async_copy(
                table_hbm.at[idx_v.at[1, pl.ds(0, cB)]],
                buf.at[pl.ds(cA, cB)], sem)
            cp0.wait()
            cp1.wait()

            for h, ngrp in ((0, cA // _L), (1, (RT - cA) // _L)):
                def grp_body(t, _, h=h):
                    base = t * _L
                    m16 = mask_v[h, pl.ds(base, _L)]
                    for r in range(_L):
                        mr = jnp.full((_L,), m16[r], jnp.float32)
                        l = h * cA + base + r
                        for j in range(nd):
                            sl = pl.ds(_L * j, _L)
                            g16 = buf[l, sl]
                            p16 = pos_v[l, sl]
                            buf[l, sl] = emb[j] + mr * (g16 - emb[j]) + p16
                    return 0

                lax.fori_loop(0, ngrp, grp_body, 0)

            pltpu.sync_copy(buf.at[pl.ds(0, L)], out_hbm.at[bb])
            return 0

        lax.fori_loop(0, bpw, chunk_body, 0)

    return sc_embed


def _pack128(x, dtype):
    B, L = x.shape
    return jnp.pad(x.astype(dtype), ((0, 0), (0, 256 - L))).reshape(2 * B, 128)


def kernel(item_id, token_mask, item_table, mask_emb, pos_emb):
    B, L = item_id.shape
    V, D = item_table.shape
    idxp = _pack128(item_id, jnp.int32)
    maskp = _pack128(token_mask, jnp.float32)
    sc_embed = _make_sc_embed(B, L, V, D)
    return sc_embed(idxp, maskp, item_table, mask_emb, pos_emb)


# speedup vs baseline: 1.4908x; 1.1020x over previous
import functools

import jax
import jax.numpy as jnp
from jax import lax
from jax.experimental import pallas as pl
from jax.experimental.pallas import tpu as pltpu
from jax.experimental.pallas import tpu_sc as plsc

_L = 16


def _make_sc_embed(B, L, V, D):
    NW = 32
    assert B % NW == 0
    bpw = B // NW
    assert bpw % 2 == 0
    nd = D // _L
    cA, cB = 128, L - 128
    RT = cA + ((cB + _L - 1) // _L) * _L
    assert 0 < cB <= 128 and cA % 8 == 0

    mesh = plsc.VectorSubcoreMesh(core_axis_name="c", subcore_axis_name="s")

    @functools.partial(
        pl.kernel,
        mesh=mesh,
        compiler_params=pltpu.CompilerParams(use_tc_tiling_on_sc=False),
        out_type=jax.ShapeDtypeStruct((B, L, D), jnp.float32),
        scratch_types=[
            pltpu.VMEM((2, 2, cA), jnp.int32),
            pltpu.VMEM((2, 2, cA), jnp.float32),
            pltpu.VMEM((RT, D), jnp.float32),
            pltpu.VMEM((1, D), jnp.float32),
            pltpu.VMEM((2, RT, D), jnp.float32),
            pltpu.SemaphoreType.DMA,
            pltpu.SemaphoreType.DMA,
            pltpu.SemaphoreType.DMA,
            pltpu.SemaphoreType.DMA,
        ],
    )
    def sc_embed(idx_hbm, mask_hbm, table_hbm, me_hbm, pos_hbm, out_hbm,
                 idx_v, mask_v, pos_v, me_v, buf, sg0, sg1, so0, so1):
        wid = lax.axis_index("s") * 2 + lax.axis_index("c")
        sg = (sg0, sg1)
        so = (so0, so1)
        pltpu.sync_copy(pos_hbm.at[pl.ds(0, cA)], pos_v.at[pl.ds(0, cA)])
        pltpu.sync_copy(pos_hbm.at[pl.ds(cA, cB)], pos_v.at[pl.ds(cA, cB)])
        pltpu.sync_copy(me_hbm, me_v)
        emb = [me_v[0, pl.ds(_L * j, _L)] for j in range(nd)]

        def stage_and_gather(g, sb):
            bb = wid * bpw + g
            pltpu.sync_copy(idx_hbm.at[pl.ds(2 * bb, 2)], idx_v.at[sb])
            pltpu.sync_copy(mask_hbm.at[pl.ds(2 * bb, 2)], mask_v.at[sb])
            pltpu.async_copy(table_hbm.at[idx_v.at[sb, 0]],
                             buf.at[sb, pl.ds(0, cA)], sg[sb])
            pltpu.async_copy(table_hbm.at[idx_v.at[sb, 1, pl.ds(0, cB)]],
                             buf.at[sb, pl.ds(cA, cB)], sg[sb])

        def wait_gather(sb):
            pltpu.make_async_copy(table_hbm.at[idx_v.at[sb, 0]],
                                  buf.at[sb, pl.ds(0, cA)], sg[sb]).wait()
            pltpu.make_async_copy(table_hbm.at[idx_v.at[sb, 1, pl.ds(0, cB)]],
                                  buf.at[sb, pl.ds(cA, cB)], sg[sb]).wait()

        def wait_out(sb, bb):
            pltpu.make_async_copy(buf.at[sb, pl.ds(0, L)],
                                  out_hbm.at[bb], so[sb]).wait()

        def compute(sb):
            for h, ngrp in ((0, cA // _L), (1, (RT - cA) // _L)):
                def grp_body(t, _, h=h):
                    base = t * _L
                    m16 = mask_v[sb, h, pl.ds(base, _L)]
                    for r in range(_L):
                        mr = jnp.full((_L,), m16[r], jnp.float32)
                        l = h * cA + base + r
                        for j in range(nd):
                            sl = pl.ds(_L * j, _L)
                            g16 = buf[sb, l, sl]
                            p16 = pos_v[l, sl]
                            buf[sb, l, sl] = emb[j] + mr * (g16 - emb[j]) + p16
                    return 0

                lax.fori_loop(0, ngrp, grp_body, 0)

        stage_and_gather(wid * bpw * 0, 0)

        def pair_body(g2, _):
            g = 2 * g2
            bb = wid * bpw + g

            @pl.when(g2 > 0)
            def _():
                wait_out(1, bb - 1)
            stage_and_gather(g + 1, 1)
            wait_gather(0)
            compute(0)
            pltpu.async_copy(buf.at[0, pl.ds(0, L)], out_hbm.at[bb], so[0])

            g = 2 * g2 + 1
            bb = wid * bpw + g
            wait_out(0, bb - 1)

            @pl.when(g2 < bpw // 2 - 1)
            def _():
                stage_and_gather(g + 1, 0)
            wait_gather(1)
            compute(1)
            pltpu.async_copy(buf.at[1, pl.ds(0, L)], out_hbm.at[bb], so[1])
            return 0

        lax.fori_loop(0, bpw // 2, pair_body, 0)
        wait_out(1, wid * bpw + bpw - 1)

    return sc_embed


def _pack128(x, dtype):
    B, L = x.shape
    return jnp.pad(x.astype(dtype), ((0, 0), (0, 256 - L))).reshape(2 * B, 128)


def kernel(item_id, token_mask, item_table, mask_emb, pos_emb):
    B, L = item_id.shape
    V, D = item_table.shape
    idxp = _pack128(item_id, jnp.int32)
    maskp = _pack128(token_mask, jnp.float32)
    sc_embed = _make_sc_embed(B, L, V, D)
    return sc_embed(idxp, maskp, item_table, mask_emb, pos_emb)
